# baseline (speedup 1.0000x reference)
import jax
import jax.numpy as jnp
from jax.experimental import pallas as pl
from jax.experimental.pallas import tpu as pltpu

_TOP_K = 8
_THRESHOLD = 1e-4
_TEMPERATURE = 0.1
_TB = 16
_CHUNKS = 1
_N = 64


def _graph_block(x_ref, o_ref):
    for chunk in range(_CHUNKS):
        _graph_chunk(x_ref, o_ref, chunk)


def _graph_chunk(x_ref, o_ref, chunk):
    x = x_ref[0, :, chunk * _TB:(chunk + 1) * _TB, :]
    xt = jnp.transpose(x, (1, 0, 2))
    xf = xt.reshape(_TB * _N, _N)
    ones_t = jnp.full((_N, _N), _TEMPERATURE, jnp.float32)
    n2t = jax.lax.dot_general(xf * xf, ones_t, (((1,), (0,)), ((), ())),
                              preferred_element_type=jnp.float32)
    xn3 = (xf * jax.lax.rsqrt(n2t)).reshape(_TB, _N, _N)
    ss = jnp.concatenate(
        [jax.lax.dot_general(
            xn3[t], xn3[t], (((1,), (1,)), ((), ())),
            preferred_element_type=jnp.float32) for t in range(_TB)],
        axis=1)
    diag = (jax.lax.broadcasted_iota(jnp.int32, (_N, _TB * _N), 0)
            == (jax.lax.broadcasted_iota(jnp.int32, (_N, _TB * _N), 1)
                & (_N - 1)))
    wk = jnp.where(diag, -30.0, ss)
    for _ in range(_TOP_K - 2):
        wk = jnp.where(wk >= jnp.max(wk, axis=0, keepdims=True), -30.0, wk)
    kth = jnp.max(wk, axis=0, keepdims=True)
    e = jnp.exp(ss)
    srow = jnp.sum(e, axis=0, keepdims=True)
    ph = e * (0.5 / srow)
    lthr = jnp.maximum(kth, jnp.log(srow) + jnp.log(_THRESHOLD))
    a = jnp.where(ss >= lthr, ph, 0.0)
    eye = jnp.eye(_N, dtype=jnp.float32)
    at = jax.lax.dot_general(a, eye, (((0,), (0,)), ((), ())),
                             preferred_element_type=jnp.float32)
    b = at.reshape(_TB, _N, _N)
    c = jnp.transpose(b, (0, 2, 1))
    o_ref[0, chunk * _TB:(chunk + 1) * _TB] = b + c


@jax.jit
def kernel(features):
    b, n, t, d = features.shape
    tblk = _TB * _CHUNKS
    grid = (b, t // tblk)
    return pl.pallas_call(
        _graph_block,
        grid=grid,
        in_specs=[pl.BlockSpec((1, n, tblk, d), lambda i, j: (i, 0, j, 0))],
        out_specs=pl.BlockSpec((1, tblk, n, n), lambda i, j: (i, j, 0, 0)),
        out_shape=jax.ShapeDtypeStruct((b, t, n, n), jnp.float32),
        compiler_params=pltpu.CompilerParams(
            dimension_semantics=("parallel", "parallel")),
    )(features)


# speedup vs baseline: 1.2853x; 1.2853x over previous
import jax
import jax.numpy as jnp
from jax.experimental import pallas as pl
from jax.experimental.pallas import tpu as pltpu

_TOP_K = 8
_THRESHOLD = 1e-4
_TEMPERATURE = 0.1
_TB = 32
_CHUNKS = 2
_N = 64


def _graph_block(x_ref, o_ref):
    for chunk in range(_CHUNKS):
        _graph_chunk(x_ref, o_ref, chunk)


def _graph_chunk(x_ref, o_ref, chunk):
    x = x_ref[0, :, chunk * _TB:(chunk + 1) * _TB, :]
    xt = jnp.transpose(x, (1, 0, 2))
    xf = xt.reshape(_TB * _N, _N)
    ones_t = jnp.full((_N, _N), _TEMPERATURE, jnp.float32)
    n2t = jax.lax.dot_general(xf * xf, ones_t, (((1,), (0,)), ((), ())),
                              preferred_element_type=jnp.float32)
    xn3 = (xf * jax.lax.rsqrt(n2t)).reshape(_TB, _N, _N)
    ss = jnp.concatenate(
        [jax.lax.dot_general(
            xn3[t], xn3[t], (((1,), (1,)), ((), ())),
            preferred_element_type=jnp.float32) for t in range(_TB)],
        axis=1)
    diag = (jax.lax.broadcasted_iota(jnp.int32, (_N, _TB * _N), 0)
            == (jax.lax.broadcasted_iota(jnp.int32, (_N, _TB * _N), 1)
                & (_N - 1)))
    wk = jnp.where(diag, -30.0, ss)
    for _ in range(_TOP_K - 2):
        wk = jnp.where(wk >= jnp.max(wk, axis=0, keepdims=True), -30.0, wk)
    kth = jnp.max(wk, axis=0, keepdims=True)
    e = jnp.exp(ss)
    srow = jnp.sum(e, axis=0, keepdims=True)
    ph = e * (0.5 / srow)
    lthr = jnp.maximum(kth, jnp.log(srow) + jnp.log(_THRESHOLD))
    a = jnp.where(ss >= lthr, ph, 0.0)
    eye = jnp.eye(_N, dtype=jnp.float32)
    at = jax.lax.dot_general(a, eye, (((0,), (0,)), ((), ())),
                             preferred_element_type=jnp.float32)
    b = at.reshape(_TB, _N, _N)
    c = jnp.transpose(b, (0, 2, 1))
    o_ref[0, chunk * _TB:(chunk + 1) * _TB] = b + c


@jax.jit
def kernel(features):
    b, n, t, d = features.shape
    tblk = _TB * _CHUNKS
    grid = (b, t // tblk)
    return pl.pallas_call(
        _graph_block,
        grid=grid,
        in_specs=[pl.BlockSpec((1, n, tblk, d), lambda i, j: (i, 0, j, 0))],
        out_specs=pl.BlockSpec((1, tblk, n, n), lambda i, j: (i, j, 0, 0)),
        out_shape=jax.ShapeDtypeStruct((b, t, n, n), jnp.float32),
        compiler_params=pltpu.CompilerParams(
            dimension_semantics=("parallel", "parallel")),
    )(features)


# speedup vs baseline: 1.2898x; 1.0035x over previous
import jax
import jax.numpy as jnp
from jax.experimental import pallas as pl
from jax.experimental.pallas import tpu as pltpu

_TOP_K = 8
_THRESHOLD = 1e-4
_TEMPERATURE = 0.1
_TB = 32
_CHUNKS = 4
_N = 64


def _graph_block(x_ref, o_ref):
    for chunk in range(_CHUNKS):
        _graph_chunk(x_ref, o_ref, chunk)


def _graph_chunk(x_ref, o_ref, chunk):
    x = x_ref[0, :, chunk * _TB:(chunk + 1) * _TB, :]
    xt = jnp.transpose(x, (1, 0, 2))
    xf = xt.reshape(_TB * _N, _N)
    ones_t = jnp.full((_N, _N), _TEMPERATURE, jnp.float32)
    n2t = jax.lax.dot_general(xf * xf, ones_t, (((1,), (0,)), ((), ())),
                              preferred_element_type=jnp.float32)
    xn3 = (xf * jax.lax.rsqrt(n2t)).reshape(_TB, _N, _N)
    ss = jnp.concatenate(
        [jax.lax.dot_general(
            xn3[t], xn3[t], (((1,), (1,)), ((), ())),
            preferred_element_type=jnp.float32) for t in range(_TB)],
        axis=1)
    diag = (jax.lax.broadcasted_iota(jnp.int32, (_N, _TB * _N), 0)
            == (jax.lax.broadcasted_iota(jnp.int32, (_N, _TB * _N), 1)
                & (_N - 1)))
    wk = jnp.where(diag, -30.0, ss)
    for _ in range(_TOP_K - 2):
        wk = jnp.where(wk >= jnp.max(wk, axis=0, keepdims=True), -30.0, wk)
    kth = jnp.max(wk, axis=0, keepdims=True)
    e = jnp.exp(ss)
    srow = jnp.sum(e, axis=0, keepdims=True)
    ph = e * (0.5 / srow)
    lthr = jnp.maximum(kth, jnp.log(srow) + jnp.log(_THRESHOLD))
    a = jnp.where(ss >= lthr, ph, 0.0)
    eye = jnp.eye(_N, dtype=jnp.float32)
    at = jax.lax.dot_general(a, eye, (((0,), (0,)), ((), ())),
                             preferred_element_type=jnp.float32)
    b = at.reshape(_TB, _N, _N)
    c = jnp.transpose(b, (0, 2, 1))
    o_ref[0, chunk * _TB:(chunk + 1) * _TB] = b + c


@jax.jit
def kernel(features):
    b, n, t, d = features.shape
    tblk = _TB * _CHUNKS
    grid = (b, t // tblk)
    return pl.pallas_call(
        _graph_block,
        grid=grid,
        in_specs=[pl.BlockSpec((1, n, tblk, d), lambda i, j: (i, 0, j, 0))],
        out_specs=pl.BlockSpec((1, tblk, n, n), lambda i, j: (i, j, 0, 0)),
        out_shape=jax.ShapeDtypeStruct((b, t, n, n), jnp.float32),
        compiler_params=pltpu.CompilerParams(
            dimension_semantics=("parallel", "parallel")),
    )(features)


# speedup vs baseline: 1.3039x; 1.0110x over previous
import jax
import jax.numpy as jnp
from jax.experimental import pallas as pl
from jax.experimental.pallas import tpu as pltpu

_TOP_K = 8
_THRESHOLD = 1e-4
_TEMPERATURE = 0.1
_TB = 32
_CHUNKS = 8
_N = 64


def _graph_block(x_ref, o_ref):
    for chunk in range(_CHUNKS):
        _graph_chunk(x_ref, o_ref, chunk)


def _graph_chunk(x_ref, o_ref, chunk):
    x = x_ref[0, :, chunk * _TB:(chunk + 1) * _TB, :]
    xt = jnp.transpose(x, (1, 0, 2))
    xf = xt.reshape(_TB * _N, _N)
    ones_t = jnp.full((_N, _N), _TEMPERATURE, jnp.float32)
    n2t = jax.lax.dot_general(xf * xf, ones_t, (((1,), (0,)), ((), ())),
                              preferred_element_type=jnp.float32)
    xn3 = (xf * jax.lax.rsqrt(n2t)).reshape(_TB, _N, _N)
    ss = jnp.concatenate(
        [jax.lax.dot_general(
            xn3[t], xn3[t], (((1,), (1,)), ((), ())),
            preferred_element_type=jnp.float32) for t in range(_TB)],
        axis=1)
    diag = (jax.lax.broadcasted_iota(jnp.int32, (_N, _TB * _N), 0)
            == (jax.lax.broadcasted_iota(jnp.int32, (_N, _TB * _N), 1)
                & (_N - 1)))
    wk = jnp.where(diag, -30.0, ss)
    for _ in range(_TOP_K - 2):
        wk = jnp.where(wk >= jnp.max(wk, axis=0, keepdims=True), -30.0, wk)
    kth = jnp.max(wk, axis=0, keepdims=True)
    e = jnp.exp(ss)
    srow = jnp.sum(e, axis=0, keepdims=True)
    ph = e * (0.5 / srow)
    lthr = jnp.maximum(kth, jnp.log(srow) + jnp.log(_THRESHOLD))
    a = jnp.where(ss >= lthr, ph, 0.0)
    eye = jnp.eye(_N, dtype=jnp.float32)
    at = jax.lax.dot_general(a, eye, (((0,), (0,)), ((), ())),
                             preferred_element_type=jnp.float32)
    b = at.reshape(_TB, _N, _N)
    c = jnp.transpose(b, (0, 2, 1))
    o_ref[0, chunk * _TB:(chunk + 1) * _TB] = b + c


@jax.jit
def kernel(features):
    b, n, t, d = features.shape
    tblk = _TB * _CHUNKS
    grid = (b, t // tblk)
    return pl.pallas_call(
        _graph_block,
        grid=grid,
        in_specs=[pl.BlockSpec((1, n, tblk, d), lambda i, j: (i, 0, j, 0))],
        out_specs=pl.BlockSpec((1, tblk, n, n), lambda i, j: (i, j, 0, 0)),
        out_shape=jax.ShapeDtypeStruct((b, t, n, n), jnp.float32),
        compiler_params=pltpu.CompilerParams(
            dimension_semantics=("parallel", "parallel")),
    )(features)


# speedup vs baseline: 2.6071x; 1.9994x over previous
import jax
import jax.numpy as jnp
from jax.experimental import pallas as pl
from jax.experimental.pallas import tpu as pltpu

_TOP_K = 8
_THRESHOLD = 1e-4
_TEMPERATURE = 0.1
_TB = 32
_CHUNKS = 16
_N = 64


def _graph_block(x_ref, o_ref):
    for chunk in range(_CHUNKS):
        _graph_chunk(x_ref, o_ref, chunk)


def _graph_chunk(x_ref, o_ref, chunk):
    x = x_ref[0, :, chunk * _TB:(chunk + 1) * _TB, :]
    xt = jnp.transpose(x, (1, 0, 2))
    xf = xt.reshape(_TB * _N, _N)
    ones_t = jnp.full((_N, _N), _TEMPERATURE, jnp.float32)
    n2t = jax.lax.dot_general(xf * xf, ones_t, (((1,), (0,)), ((), ())),
                              preferred_element_type=jnp.float32)
    xn3 = (xf * jax.lax.rsqrt(n2t)).reshape(_TB, _N, _N)
    ss = jnp.concatenate(
        [jax.lax.dot_general(
            xn3[t], xn3[t], (((1,), (1,)), ((), ())),
            preferred_element_type=jnp.float32) for t in range(_TB)],
        axis=1)
    diag = (jax.lax.broadcasted_iota(jnp.int32, (_N, _TB * _N), 0)
            == (jax.lax.broadcasted_iota(jnp.int32, (_N, _TB * _N), 1)
                & (_N - 1)))
    wk = jnp.where(diag, -30.0, ss)
    for _ in range(_TOP_K - 2):
        wk = jnp.where(wk >= jnp.max(wk, axis=0, keepdims=True), -30.0, wk)
    kth = jnp.max(wk, axis=0, keepdims=True)
    e = jnp.exp(ss)
    srow = jnp.sum(e, axis=0, keepdims=True)
    ph = e * (0.5 / srow)
    lthr = jnp.maximum(kth, jnp.log(srow) + jnp.log(_THRESHOLD))
    a = jnp.where(ss >= lthr, ph, 0.0)
    eye = jnp.eye(_N, dtype=jnp.float32)
    at = jax.lax.dot_general(a, eye, (((0,), (0,)), ((), ())),
                             preferred_element_type=jnp.float32)
    b = at.reshape(_TB, _N, _N)
    c = jnp.transpose(b, (0, 2, 1))
    o_ref[0, chunk * _TB:(chunk + 1) * _TB] = b + c


@jax.jit
def kernel(features):
    b, n, t, d = features.shape
    tblk = _TB * _CHUNKS
    grid = (b, t // tblk)
    return pl.pallas_call(
        _graph_block,
        grid=grid,
        in_specs=[pl.BlockSpec((1, n, tblk, d), lambda i, j: (i, 0, j, 0))],
        out_specs=pl.BlockSpec((1, tblk, n, n), lambda i, j: (i, j, 0, 0)),
        out_shape=jax.ShapeDtypeStruct((b, t, n, n), jnp.float32),
        compiler_params=pltpu.CompilerParams(
            dimension_semantics=("parallel", "parallel")),
    )(features)
